# Initial kernel scaffold; baseline (speedup 1.0000x reference)
import jax
import jax.numpy as jnp
from jax.experimental import pallas as pl


def kernel(x, norm_weight, gamma):
    raise NotImplementedError("write your pallas kernel here")


# speedup vs baseline: 11.0532x; 11.0532x over previous
import functools

import jax
import jax.numpy as jnp
from jax.experimental import pallas as pl
from jax.experimental.pallas import tpu as pltpu

D_MODEL = 2048
K = 32
EPS = 1e-5
ROWS_PER_BLOCK = 256
N_ITERS = 31


def _topk_mask_body(x_ref, w_ref, g_ref, o_ref):
    xm = x_ref[...]
    w = w_ref[...]
    g = g_ref[...]
    mean = jnp.mean(xm, axis=1, keepdims=True)
    xc = xm - mean
    var = jnp.mean(xc * xc, axis=1, keepdims=True)
    rstd = jax.lax.rsqrt(var + EPS)
    xn = xc * rstd * w
    bits = jax.lax.bitcast_convert_type(xn, jnp.int32) & jnp.int32(0x7FFFFFFF)
    hi = jnp.max(bits, axis=1, keepdims=True)
    lo = jnp.zeros_like(hi)

    def body(_, carry):
        lo, hi = carry
        mid = lo + ((hi - lo + 1) >> 1)
        cnt = jnp.sum((bits >= mid).astype(jnp.int32), axis=1, keepdims=True)
        ok = cnt >= K
        return jnp.where(ok, mid, lo), jnp.where(ok, hi, mid - 1)

    lo, hi = jax.lax.fori_loop(0, N_ITERS, body, (lo, hi))
    keep = bits >= lo
    o_ref[...] = xm + jnp.where(keep, xn * g, 0.0)


@jax.jit
def kernel(x, norm_weight, gamma):
    B, S, D = x.shape
    rows = B * S
    x2 = x.reshape(rows, D)
    grid = (rows // ROWS_PER_BLOCK,)
    out = pl.pallas_call(
        _topk_mask_body,
        grid=grid,
        in_specs=[
            pl.BlockSpec((ROWS_PER_BLOCK, D), lambda i: (i, 0)),
            pl.BlockSpec((1, D), lambda i: (0, 0)),
            pl.BlockSpec((1, D), lambda i: (0, 0)),
        ],
        out_specs=pl.BlockSpec((ROWS_PER_BLOCK, D), lambda i: (i, 0)),
        out_shape=jax.ShapeDtypeStruct((rows, D), x.dtype),
        compiler_params=pltpu.CompilerParams(
            dimension_semantics=("arbitrary",),
        ),
    )(x2, norm_weight.reshape(1, D), gamma.reshape(1, D))
    return out.reshape(B, S, D)
